# baseline (speedup 1.0000x reference)
import jax
import jax.numpy as jnp
from jax.experimental import pallas as pl
from jax.experimental.pallas import tpu as pltpu

_NCLS = 15
_NCH = _NCLS + 5
_G = 52 * 52
_SCALE = 8.0
_H = 2
_HW = _H * 52


def _decode(t):
    xy = jax.nn.sigmoid(t[0:2]) * _SCALE
    wh = t[2:4] * _SCALE
    conf = jax.nn.sigmoid(t[4:5])
    logits = t[5:_NCH]
    m = jnp.max(logits, axis=0, keepdims=True)
    e = jnp.exp(logits - m)
    cls = e / jnp.sum(e, axis=0, keepdims=True)
    return jnp.concatenate([xy, wh, conf, cls], axis=0)


def _body(x_ref, o_ref):
    v = x_ref[...]
    nb = v.shape[-1]
    t = jnp.transpose(v.reshape(_HW, 3 * _NCH, nb), (1, 0, 2))
    for a in range(3):
        o_ref[:, a] = _decode(t[a * _NCH:(a + 1) * _NCH])


def kernel(x):
    nB = x.shape[0]
    xt = jnp.transpose(x, (2, 3, 1, 0))
    out = pl.pallas_call(
        _body,
        grid=(52 // _H,),
        in_specs=[pl.BlockSpec((_H, 52, 3 * _NCH, nB), lambda h: (h, 0, 0, 0))],
        out_specs=pl.BlockSpec((_NCH, 3, _HW, nB), lambda h: (0, 0, h, 0)),
        out_shape=jax.ShapeDtypeStruct((_NCH, 3, _G, nB), jnp.float32),
        compiler_params=pltpu.CompilerParams(
            dimension_semantics=("parallel",),
        ),
    )(xt)
    return jnp.transpose(out.reshape(_NCH, 3 * _G, nB), (2, 1, 0))


# speedup vs baseline: 1.0843x; 1.0843x over previous
import jax
import jax.numpy as jnp
from jax.experimental import pallas as pl
from jax.experimental.pallas import tpu as pltpu

_NCLS = 15
_NCH = _NCLS + 5
_G = 52 * 52
_SCALE = 8.0
_H = 4
_HW = _H * 52


def _decode(t):
    xy = jax.nn.sigmoid(t[0:2]) * _SCALE
    wh = t[2:4] * _SCALE
    conf = jax.nn.sigmoid(t[4:5])
    logits = t[5:_NCH]
    m = jnp.max(logits, axis=0, keepdims=True)
    e = jnp.exp(logits - m)
    cls = e / jnp.sum(e, axis=0, keepdims=True)
    return jnp.concatenate([xy, wh, conf, cls], axis=0)


def _body(x_ref, o_ref):
    v = x_ref[...]
    nb = v.shape[-1]
    t = jnp.transpose(v.reshape(_HW, 3 * _NCH, nb), (1, 0, 2))
    for a in range(3):
        o_ref[:, a] = _decode(t[a * _NCH:(a + 1) * _NCH])


def kernel(x):
    nB = x.shape[0]
    xt = jnp.transpose(x, (2, 3, 1, 0))
    out = pl.pallas_call(
        _body,
        grid=(52 // _H,),
        in_specs=[pl.BlockSpec((_H, 52, 3 * _NCH, nB), lambda h: (h, 0, 0, 0))],
        out_specs=pl.BlockSpec((_NCH, 3, _HW, nB), lambda h: (0, 0, h, 0)),
        out_shape=jax.ShapeDtypeStruct((_NCH, 3, _G, nB), jnp.float32),
        compiler_params=pltpu.CompilerParams(
            dimension_semantics=("parallel",),
        ),
    )(xt)
    return jnp.transpose(out.reshape(_NCH, 3 * _G, nB), (2, 1, 0))
